# Initial kernel scaffold; baseline (speedup 1.0000x reference)
import jax
import jax.numpy as jnp
from jax.experimental import pallas as pl


def kernel(x, edge_index, params):
    raise NotImplementedError("write your pallas kernel here")


# speedup vs baseline: 6.3171x; 6.3171x over previous
import functools

import jax
import jax.numpy as jnp
from jax import lax
from jax.experimental import pallas as pl
from jax.experimental.pallas import tpu as pltpu
from jax.experimental.pallas import tpu_sc as plsc

_N = 10000
_H = 128
_C = 40
_L = 3
_E = 320000

_NC = 2
_NS = 16
_EPW = _E // (_NC * _NS)
_CH = 80
_NCH = _EPW // _CH
_NPAD = 10240
_RPT = _NPAD // _NS
_IB = 25
_NG = _NCH // _IB


def _seg_sum_partials(h, src, dst):
    mesh = plsc.VectorSubcoreMesh(core_axis_name="c", subcore_axis_name="s")

    @functools.partial(
        pl.kernel,
        mesh=mesh,
        out_type=jax.ShapeDtypeStruct((_NC, _NPAD, _H), jnp.float32),
        scratch_types=[
            pltpu.VMEM((_IB, _CH), jnp.int32),
            pltpu.VMEM((_IB, _CH), jnp.int32),
            pltpu.VMEM((_CH, _H), jnp.float32),
            pltpu.VMEM_SHARED((_NPAD, _H), jnp.float32),
            pltpu.SemaphoreType.DMA,
        ],
    )
    def seg(h_hbm, src_hbm, dst_hbm, out_hbm, src_v, dst_v, rows_v, acc, sem):
        c = lax.axis_index("c")
        s = lax.axis_index("s")

        def zf(i, carry):
            rows_v[i // 8, pl.ds((i % 8) * 16, 16)] = jnp.zeros((16,), jnp.float32)
            return carry

        lax.fori_loop(0, _CH * 8, zf, 0)

        row0 = s * _RPT
        for k in range(_RPT // _CH):
            pltpu.sync_copy(rows_v, acc.at[pl.ds(row0 + k * _CH, _CH)])
        plsc.subcore_barrier()

        def outer(b, carry):
            pltpu.sync_copy(src_hbm.at[c, s, b], src_v)
            pltpu.sync_copy(dst_hbm.at[c, s, b], dst_v)

            def body(j, inner):
                pltpu.async_copy(h_hbm.at[src_v.at[j]], rows_v, sem).wait()
                pltpu.sync_copy(rows_v, acc.at[dst_v.at[j]], add=True)
                return inner

            lax.fori_loop(0, _IB, body, 0)
            return carry

        lax.fori_loop(0, _NG, outer, 0)

        plsc.subcore_barrier()
        pltpu.sync_copy(acc.at[pl.ds(row0, _RPT)], out_hbm.at[c, pl.ds(row0, _RPT)])

    return seg(h, src, dst)


def _mlp_body(h_ref, p_ref, eps_ref, w1_ref, b1_ref, g1_ref, be1_ref,
              w2_ref, b2_ref, g2_ref, be2_ref, o_ref):
    u = (1.0 + eps_ref[0, 0]) * h_ref[...] + p_ref[0, :_N] + p_ref[1, :_N]
    h1 = jnp.dot(u, w1_ref[...], preferred_element_type=jnp.float32) + b1_ref[...]
    h1 = jnp.maximum(h1, 0.0)
    mu = jnp.mean(h1, axis=0, keepdims=True)
    var = jnp.mean((h1 - mu) ** 2, axis=0, keepdims=True)
    h1 = g1_ref[...] * (h1 - mu) * lax.rsqrt(var + 1e-5) + be1_ref[...]
    h2 = jnp.dot(h1, w2_ref[...], preferred_element_type=jnp.float32) + b2_ref[...]
    h2 = jnp.maximum(h2, 0.0)
    mu2 = jnp.mean(h2, axis=0, keepdims=True)
    var2 = jnp.mean((h2 - mu2) ** 2, axis=0, keepdims=True)
    o_ref[...] = g2_ref[...] * (h2 - mu2) * lax.rsqrt(var2 + 1e-5) + be2_ref[...]


def _mlp_tc(h, parts, eps, w1, b1, g1, be1, w2, b2, g2, be2):
    return pl.pallas_call(
        _mlp_body,
        out_shape=jax.ShapeDtypeStruct((_N, _H), jnp.float32),
    )(h, parts, eps.reshape(1, 1), w1, b1.reshape(1, _H), g1.reshape(1, _H),
      be1.reshape(1, _H), w2, b2.reshape(1, _H), g2.reshape(1, _H),
      be2.reshape(1, _H))


def _head_body(h_ref, w1_ref, b1_ref, w2_ref, b2_ref, o_ref):
    z = jnp.dot(h_ref[...], w1_ref[...], preferred_element_type=jnp.float32)
    z = jnp.maximum(z + b1_ref[...], 0.0)
    z = jnp.dot(z, w2_ref[...], preferred_element_type=jnp.float32) + b2_ref[...]
    m = jnp.max(z, axis=-1, keepdims=True)
    e = z - m
    o_ref[...] = e - jnp.log(jnp.sum(jnp.exp(e), axis=-1, keepdims=True))


def _head_tc(h, w1, b1, w2, b2):
    return pl.pallas_call(
        _head_body,
        out_shape=jax.ShapeDtypeStruct((_N, _C), jnp.float32),
    )(h, w1, b1.reshape(1, _H), w2, b2.reshape(1, _C))


def kernel(x, edge_index, params):
    src = edge_index[0].astype(jnp.int32).reshape(_NC, _NS, _NG, _IB, _CH)
    dst = edge_index[1].astype(jnp.int32).reshape(_NC, _NS, _NG, _IB, _CH)
    h = x
    for l in range(_L):
        pre = "conv%d" % l
        parts = _seg_sum_partials(h, src, dst)
        h = _mlp_tc(
            h, parts, params[pre + "_eps"],
            params[pre + "_W1"], params[pre + "_b1"],
            params[pre + "_g1"], params[pre + "_be1"],
            params[pre + "_W2"], params[pre + "_b2"],
            params[pre + "_g2"], params[pre + "_be2"],
        )
    return _head_tc(h, params["lin1_W"], params["lin1_b"],
                    params["lin2_W"], params["lin2_b"])


# speedup vs baseline: 8.1883x; 1.2962x over previous
import functools

import jax
import jax.numpy as jnp
from jax import lax
from jax.experimental import pallas as pl
from jax.experimental.pallas import tpu as pltpu
from jax.experimental.pallas import tpu_sc as plsc

_N = 10000
_H = 128
_C = 40
_L = 3
_E = 320000

_NC = 2
_NS = 16
_EPW = _E // (_NC * _NS)
_CH = 80
_NCH = _EPW // _CH
_NPAD = 10240
_RPT = _NPAD // _NS


def _seg_sum_partials(h, src, dst):
    mesh = plsc.VectorSubcoreMesh(core_axis_name="c", subcore_axis_name="s")

    @functools.partial(
        pl.kernel,
        mesh=mesh,
        out_type=jax.ShapeDtypeStruct((_NC, _NPAD, _H), jnp.float32),
        scratch_types=[
            pltpu.VMEM((_EPW,), jnp.int32),
            pltpu.VMEM((_NCH, _CH), jnp.int32),
            pltpu.VMEM((_CH, _H), jnp.float32),
            pltpu.VMEM((_CH, _H), jnp.float32),
            pltpu.VMEM_SHARED((_NPAD, _H), jnp.float32),
            pltpu.SemaphoreType.DMA,
            pltpu.SemaphoreType.DMA,
        ],
    )
    def seg(h_hbm, src_hbm, dst_hbm, out_hbm, src_v, dst_v, rows0, rows1, acc,
            sem0, sem1):
        c = lax.axis_index("c")
        s = lax.axis_index("s")

        def zf(i, carry):
            rows0[i // 8, pl.ds((i % 8) * 16, 16)] = jnp.zeros((16,), jnp.float32)
            return carry

        lax.fori_loop(0, _CH * 8, zf, 0)

        row0 = s * _RPT
        for k in range(_RPT // _CH):
            pltpu.sync_copy(rows0, acc.at[pl.ds(row0 + k * _CH, _CH)])

        pltpu.sync_copy(src_hbm.at[c, s], src_v)
        pltpu.sync_copy(dst_hbm.at[c, s], dst_v)
        plsc.subcore_barrier()

        def sidx(j):
            return src_v.at[pl.ds(j * _CH, _CH)]

        pltpu.async_copy(h_hbm.at[sidx(0)], rows0, sem0)

        def body(k, carry):
            j = 2 * k
            pltpu.make_async_copy(h_hbm.at[sidx(j)], rows0, sem0).wait()
            pltpu.async_copy(h_hbm.at[sidx(j + 1)], rows1, sem1)
            pltpu.sync_copy(rows0, acc.at[dst_v.at[j]], add=True)
            pltpu.make_async_copy(h_hbm.at[sidx(j + 1)], rows1, sem1).wait()
            pltpu.async_copy(h_hbm.at[sidx(j + 2)], rows0, sem0)
            pltpu.sync_copy(rows1, acc.at[dst_v.at[j + 1]], add=True)
            return carry

        lax.fori_loop(0, (_NCH - 1) // 2, body, 0)
        pltpu.make_async_copy(h_hbm.at[sidx(_NCH - 1)], rows0, sem0).wait()
        pltpu.sync_copy(rows0, acc.at[dst_v.at[_NCH - 1]], add=True)

        plsc.subcore_barrier()
        pltpu.sync_copy(acc.at[pl.ds(row0, _RPT)], out_hbm.at[c, pl.ds(row0, _RPT)])

    return seg(h, src, dst)


def _mlp_body(h_ref, p_ref, eps_ref, w1_ref, b1_ref, g1_ref, be1_ref,
              w2_ref, b2_ref, g2_ref, be2_ref, o_ref):
    u = (1.0 + eps_ref[0, 0]) * h_ref[...] + p_ref[0, :_N] + p_ref[1, :_N]
    h1 = jnp.dot(u, w1_ref[...], preferred_element_type=jnp.float32) + b1_ref[...]
    h1 = jnp.maximum(h1, 0.0)
    mu = jnp.mean(h1, axis=0, keepdims=True)
    var = jnp.mean((h1 - mu) ** 2, axis=0, keepdims=True)
    h1 = g1_ref[...] * (h1 - mu) * lax.rsqrt(var + 1e-5) + be1_ref[...]
    h2 = jnp.dot(h1, w2_ref[...], preferred_element_type=jnp.float32) + b2_ref[...]
    h2 = jnp.maximum(h2, 0.0)
    mu2 = jnp.mean(h2, axis=0, keepdims=True)
    var2 = jnp.mean((h2 - mu2) ** 2, axis=0, keepdims=True)
    o_ref[...] = g2_ref[...] * (h2 - mu2) * lax.rsqrt(var2 + 1e-5) + be2_ref[...]


def _mlp_tc(h, parts, eps, w1, b1, g1, be1, w2, b2, g2, be2):
    return pl.pallas_call(
        _mlp_body,
        out_shape=jax.ShapeDtypeStruct((_N, _H), jnp.float32),
    )(h, parts, eps.reshape(1, 1), w1, b1.reshape(1, _H), g1.reshape(1, _H),
      be1.reshape(1, _H), w2, b2.reshape(1, _H), g2.reshape(1, _H),
      be2.reshape(1, _H))


def _head_body(h_ref, w1_ref, b1_ref, w2_ref, b2_ref, o_ref):
    z = jnp.dot(h_ref[...], w1_ref[...], preferred_element_type=jnp.float32)
    z = jnp.maximum(z + b1_ref[...], 0.0)
    z = jnp.dot(z, w2_ref[...], preferred_element_type=jnp.float32) + b2_ref[...]
    m = jnp.max(z, axis=-1, keepdims=True)
    e = z - m
    o_ref[...] = e - jnp.log(jnp.sum(jnp.exp(e), axis=-1, keepdims=True))


def _head_tc(h, w1, b1, w2, b2):
    return pl.pallas_call(
        _head_body,
        out_shape=jax.ShapeDtypeStruct((_N, _C), jnp.float32),
    )(h, w1, b1.reshape(1, _H), w2, b2.reshape(1, _C))


def kernel(x, edge_index, params):
    src = edge_index[0].astype(jnp.int32).reshape(_NC, _NS, _EPW)
    dst = edge_index[1].astype(jnp.int32).reshape(_NC, _NS, _NCH, _CH)
    h = x
    for l in range(_L):
        pre = "conv%d" % l
        parts = _seg_sum_partials(h, src, dst)
        h = _mlp_tc(
            h, parts, params[pre + "_eps"],
            params[pre + "_W1"], params[pre + "_b1"],
            params[pre + "_g1"], params[pre + "_be1"],
            params[pre + "_W2"], params[pre + "_b2"],
            params[pre + "_g2"], params[pre + "_be2"],
        )
    return _head_tc(h, params["lin1_W"], params["lin1_b"],
                    params["lin2_W"], params["lin2_b"])
